# baseline (speedup 1.0000x reference)
import functools

import jax
import jax.numpy as jnp
from jax import lax
from jax.experimental import pallas as pl
from jax.experimental.pallas import tpu as pltpu
from jax.experimental.pallas import tpu_sc as plsc

NBINS = 2048
HSTRIDE = NBINS + 3
NHCOPY = 2
NC = 2
NS = 16
NLANE = 16
NW = NC * NS

N_TOTAL = 2 * 8192 * 4096
N_ROWS = 16384
N_COLS = 4096
ROWS_W = N_ROWS // NW
CHUNK_R = 8
NCHUNK = ROWS_W // CHUNK_R
HB_COLS = N_COLS // 2
HB_NCHUNK = 2 * NCHUNK


_MM_ROWS = 16384
_MM_BM = 512
_MM_GRID = _MM_ROWS // _MM_BM


def _minmax_body(x_ref, mn_ref, mx_ref):
    i = pl.program_id(0)

    @pl.when(i == 0)
    def _():
        mn_ref[0, 0] = jnp.float32(jnp.inf)
        mx_ref[0, 0] = jnp.float32(-jnp.inf)

    blk = x_ref[...]
    mn_ref[0, 0] = jnp.minimum(mn_ref[0, 0], jnp.min(blk))
    mx_ref[0, 0] = jnp.maximum(mx_ref[0, 0], jnp.max(blk))


def _tc_minmax(x2d):
    return pl.pallas_call(
        _minmax_body,
        grid=(_MM_GRID,),
        in_specs=[pl.BlockSpec((_MM_BM, 4096), lambda i: (i, 0))],
        out_specs=[
            pl.BlockSpec(memory_space=pltpu.SMEM),
            pl.BlockSpec(memory_space=pltpu.SMEM),
        ],
        out_shape=[
            jax.ShapeDtypeStruct((1, 1), jnp.float32),
            jax.ShapeDtypeStruct((1, 1), jnp.float32),
        ],
    )(x2d)


def _mm_body(x_hbm, out_hbm, buf0, buf1, res, sem0, sem1):
    c = lax.axis_index("c")
    s = lax.axis_index("s")
    wid = s * NC + c
    base = wid * ROWS_W

    def cp(ch, buf, sem):
        return pltpu.make_async_copy(
            x_hbm.at[pl.ds((base + ch * CHUNK_R), CHUNK_R), :], buf, sem)

    cp(0, buf0, sem0).start()
    cp(1, buf1, sem1).start()

    pos = jnp.full((NLANE,), jnp.inf, jnp.float32)
    neg = jnp.full((NLANE,), -jnp.inf, jnp.float32)

    def compute(buf, acc):
        for r in range(CHUNK_R):
            def body(i, a, _r=r):
                mns, mxs = a
                mns, mxs = list(mns), list(mxs)
                for k in range(4):
                    v = buf[_r, pl.ds((i * 4 + k) * NLANE, NLANE)]
                    mns[k] = jnp.minimum(mns[k], v)
                    mxs[k] = jnp.maximum(mxs[k], v)
                return tuple(mns), tuple(mxs)

            acc = lax.fori_loop(0, N_COLS // (4 * NLANE), body, acc,
                                unroll=2)
        return acc

    def pair(p, acc):
        a = 2 * p
        cp(a, buf0, sem0).wait()
        acc = compute(buf0, acc)

        @pl.when(a + 2 < NCHUNK)
        def _():
            cp(a + 2, buf0, sem0).start()

        cp(a + 1, buf1, sem1).wait()
        acc = compute(buf1, acc)

        @pl.when(a + 3 < NCHUNK)
        def _():
            cp(a + 3, buf1, sem1).start()

        return acc

    acc0 = ((pos, pos, pos, pos), (neg, neg, neg, neg))
    (mns, mxs) = lax.fori_loop(0, NCHUNK // 2, pair, acc0)
    mn = jnp.minimum(jnp.minimum(mns[0], mns[1]),
                     jnp.minimum(mns[2], mns[3]))
    mx = jnp.maximum(jnp.maximum(mxs[0], mxs[1]),
                     jnp.maximum(mxs[2], mxs[3]))
    res[pl.ds(0, NLANE)] = mn
    res[pl.ds(NLANE, NLANE)] = mx
    pltpu.sync_copy(res.at[pl.ds(0, NLANE)],
                    out_hbm.at[pl.ds(wid * NLANE, NLANE)])
    pltpu.sync_copy(res.at[pl.ds(NLANE, NLANE)],
                    out_hbm.at[pl.ds((NW + wid) * NLANE, NLANE)])


_sc_minmax = functools.partial(
    pl.kernel,
    out_type=jax.ShapeDtypeStruct((2 * NW * NLANE,), jnp.float32),
    mesh=plsc.VectorSubcoreMesh(core_axis_name="c", subcore_axis_name="s"),
    scratch_types=[
        pltpu.VMEM((CHUNK_R, N_COLS), jnp.float32),
        pltpu.VMEM((CHUNK_R, N_COLS), jnp.float32),
        pltpu.VMEM((2 * NLANE,), jnp.float32),
        pltpu.SemaphoreType.DMA,
        pltpu.SemaphoreType.DMA,
    ],
    compiler_params=pltpu.CompilerParams(needs_layout_passes=False),
)(_mm_body)


def _hist_body(x_hbm, mmp_hbm, out_hbm,
               buf0, buf1, mm_buf, histf, histr,
               shared, sem0, sem1):
    c = lax.axis_index("c")
    s = lax.axis_index("s")
    wid = s * NC + c
    base = wid * ROWS_W

    pltpu.sync_copy(mmp_hbm, mm_buf)
    mnv = mm_buf[pl.ds(0, NLANE)]
    mxv = mm_buf[pl.ds(NW * NLANE, NLANE)]
    for i in range(1, NW):
        mnv = jnp.minimum(mnv, mm_buf[pl.ds(i * NLANE, NLANE)])
        mxv = jnp.maximum(mxv, mm_buf[pl.ds((NW + i) * NLANE, NLANE)])
    mn_s = jnp.min(mnv)
    mx_s = jnp.max(mxv)
    mn_vec = jnp.full((NLANE,), mn_s, jnp.float32)
    w_vec = (jnp.full((NLANE,), mx_s, jnp.float32) - mn_vec) * (1.0 / NBINS)
    safe_w = jnp.where(w_vec == 0.0, jnp.float32(1.0), w_vec)
    inv_vec = jnp.float32(1.0) / safe_w

    zero16 = jnp.zeros((NLANE,), jnp.float32)
    ones16 = jnp.ones((NLANE,), jnp.float32)
    lane_off = lax.iota(jnp.int32, NLANE) * HSTRIDE
    copy_off = [lane_off + cc * (NLANE * HSTRIDE) for cc in range(NHCOPY)]

    def zbody(i, carry):
        histf[pl.ds(i * NLANE, NLANE)] = zero16
        return carry

    lax.fori_loop(0, NHCOPY * NLANE * HSTRIDE // NLANE, zbody, 0)

    def cp(ch, buf, sem):
        band = lax.rem(ch, NCHUNK)
        colh = lax.div(ch, NCHUNK)
        return pltpu.make_async_copy(
            x_hbm.at[pl.ds((base + band * CHUNK_R), CHUNK_R),
                     pl.ds(colh * HB_COLS, HB_COLS)], buf, sem)

    cp(0, buf0, sem0).start()
    cp(1, buf1, sem1).start()

    def compute(buf):
        for r in range(CHUNK_R):
            @plsc.parallel_loop(0, HB_COLS // NLANE, NHCOPY, unroll=4)
            def _(i, _r=r):
                for cc in range(NHCOPY):
                    v = buf[_r, pl.ds((i + cc) * NLANE, NLANE)]
                    t = (v - mn_vec) * inv_vec
                    idx = t.astype(jnp.int32)
                    plsc.addupdate_scatter(histf, [idx + copy_off[cc]],
                                           ones16)

    def pair(p, carry):
        a = 2 * p
        cp(a, buf0, sem0).wait()
        compute(buf0)

        @pl.when(a + 2 < HB_NCHUNK)
        def _():
            cp(a + 2, buf0, sem0).start()

        cp(a + 1, buf1, sem1).wait()
        compute(buf1)

        @pl.when(a + 3 < HB_NCHUNK)
        def _():
            cp(a + 3, buf1, sem1).start()

        return carry

    lax.fori_loop(0, HB_NCHUNK // 2, pair, 0)

    for cc in range(NHCOPY):
        ov = zero16
        for k in range(NBINS, HSTRIDE):
            ov = ov + plsc.load_gather(histf, [copy_off[cc] + k])
        last = plsc.load_gather(histf, [copy_off[cc] + (NBINS - 1)])
        plsc.store_scatter(histf, [copy_off[cc] + (NBINS - 1)], last + ov)

    def rbody(j, carry):
        col = j * NLANE
        acc = zero16
        for l in range(NLANE * NHCOPY):
            acc = acc + histf[pl.ds(l * HSTRIDE + col, NLANE)]
        histr[pl.ds(col, NLANE)] = acc
        return carry

    lax.fori_loop(0, NBINS // NLANE, rbody, 0)

    pltpu.sync_copy(histr, shared.at[s])
    plsc.subcore_barrier()

    STRIPE = NBINS // NS
    for l in range(NS):
        pltpu.sync_copy(shared.at[l, pl.ds(s * STRIPE, STRIPE)],
                        buf0.at[0, pl.ds(l * STRIPE, STRIPE)])

    def sbody(j, carry):
        col = j * NLANE
        acc = zero16
        for l in range(NS):
            acc = acc + buf0[0, pl.ds(l * STRIPE + col, NLANE)]
        histr[pl.ds(col, NLANE)] = acc
        return carry

    lax.fori_loop(0, STRIPE // NLANE, sbody, 0)

    pltpu.sync_copy(histr.at[pl.ds(0, STRIPE)],
                    out_hbm.at[c, pl.ds(s * STRIPE, STRIPE)])


_sc_hist = functools.partial(
    pl.kernel,
    out_type=jax.ShapeDtypeStruct((NC, NBINS), jnp.float32),
    mesh=plsc.VectorSubcoreMesh(core_axis_name="c", subcore_axis_name="s"),
    scratch_types=[
        pltpu.VMEM((CHUNK_R, HB_COLS), jnp.float32),
        pltpu.VMEM((CHUNK_R, HB_COLS), jnp.float32),
        pltpu.VMEM((2 * NW * NLANE,), jnp.float32),
        pltpu.VMEM((NHCOPY * NLANE * HSTRIDE,), jnp.float32),
        pltpu.VMEM((NBINS,), jnp.float32),
        pltpu.VMEM_SHARED((NS, NBINS), jnp.float32),
        pltpu.SemaphoreType.DMA,
        pltpu.SemaphoreType.DMA,
    ],
    compiler_params=pltpu.CompilerParams(needs_layout_passes=False),
)(_hist_body)


def _final_body(p_ref, mm_ref, h_ref, mn_ref, mx_ref):
    h_ref[...] = p_ref[0:1, :] + p_ref[1:2, :]
    mn_ref[0, 0] = jnp.min(mm_ref[0:1, :])
    mx_ref[0, 0] = jnp.max(mm_ref[1:2, :])


def _tc_finalize(partials, mmp):
    return pl.pallas_call(
        _final_body,
        out_specs=[
            pl.BlockSpec(memory_space=pltpu.VMEM),
            pl.BlockSpec(memory_space=pltpu.SMEM),
            pl.BlockSpec(memory_space=pltpu.SMEM),
        ],
        out_shape=[
            jax.ShapeDtypeStruct((1, NBINS), jnp.float32),
            jax.ShapeDtypeStruct((1, 1), jnp.float32),
            jax.ShapeDtypeStruct((1, 1), jnp.float32),
        ],
    )(partials, mmp.reshape(2, NW * NLANE))


def kernel(x):
    x2d = x.reshape(N_ROWS, N_COLS)
    mmp = _sc_minmax(x2d)
    partials = _sc_hist(x2d, mmp)
    hist2d, mn11, mx11 = _tc_finalize(partials, mmp)
    return x, hist2d.reshape(NBINS), mn11.reshape(()), mx11.reshape(())


# speedup vs baseline: 1.0472x; 1.0472x over previous
import functools

import jax
import jax.numpy as jnp
from jax import lax
from jax.experimental import pallas as pl
from jax.experimental.pallas import tpu as pltpu
from jax.experimental.pallas import tpu_sc as plsc

NBINS = 2048
HSTRIDE = NBINS + 3
NHCOPY = 1
NC = 2
NS = 16
NLANE = 16
NW = NC * NS

N_TOTAL = 2 * 8192 * 4096
N_ROWS = 16384
N_COLS = 4096
ROWS_W = N_ROWS // NW
CHUNK_R = 8
NCHUNK = ROWS_W // CHUNK_R
HB_COLS = N_COLS
HB_NCHUNK = NCHUNK


_MM_ROWS = 16384
_MM_BM = 512
_MM_GRID = _MM_ROWS // _MM_BM


def _minmax_body(x_ref, mn_ref, mx_ref):
    i = pl.program_id(0)

    @pl.when(i == 0)
    def _():
        mn_ref[0, 0] = jnp.float32(jnp.inf)
        mx_ref[0, 0] = jnp.float32(-jnp.inf)

    blk = x_ref[...]
    mn_ref[0, 0] = jnp.minimum(mn_ref[0, 0], jnp.min(blk))
    mx_ref[0, 0] = jnp.maximum(mx_ref[0, 0], jnp.max(blk))


def _tc_minmax(x2d):
    return pl.pallas_call(
        _minmax_body,
        grid=(_MM_GRID,),
        in_specs=[pl.BlockSpec((_MM_BM, 4096), lambda i: (i, 0))],
        out_specs=[
            pl.BlockSpec(memory_space=pltpu.SMEM),
            pl.BlockSpec(memory_space=pltpu.SMEM),
        ],
        out_shape=[
            jax.ShapeDtypeStruct((1, 1), jnp.float32),
            jax.ShapeDtypeStruct((1, 1), jnp.float32),
        ],
    )(x2d)


def _mm_body(x_hbm, out_hbm, buf0, buf1, res, sem0, sem1):
    c = lax.axis_index("c")
    s = lax.axis_index("s")
    wid = s * NC + c
    base = wid * ROWS_W

    def cp(ch, buf, sem):
        return pltpu.make_async_copy(
            x_hbm.at[pl.ds((base + ch * CHUNK_R), CHUNK_R), :], buf, sem)

    cp(0, buf0, sem0).start()
    cp(1, buf1, sem1).start()

    pos = jnp.full((NLANE,), jnp.inf, jnp.float32)
    neg = jnp.full((NLANE,), -jnp.inf, jnp.float32)

    def compute(buf, acc):
        for r in range(CHUNK_R):
            def body(i, a, _r=r):
                mns, mxs = a
                mns, mxs = list(mns), list(mxs)
                for k in range(4):
                    v = buf[_r, pl.ds((i * 4 + k) * NLANE, NLANE)]
                    mns[k] = jnp.minimum(mns[k], v)
                    mxs[k] = jnp.maximum(mxs[k], v)
                return tuple(mns), tuple(mxs)

            acc = lax.fori_loop(0, N_COLS // (4 * NLANE), body, acc,
                                unroll=2)
        return acc

    def pair(p, acc):
        a = 2 * p
        cp(a, buf0, sem0).wait()
        acc = compute(buf0, acc)

        @pl.when(a + 2 < NCHUNK)
        def _():
            cp(a + 2, buf0, sem0).start()

        cp(a + 1, buf1, sem1).wait()
        acc = compute(buf1, acc)

        @pl.when(a + 3 < NCHUNK)
        def _():
            cp(a + 3, buf1, sem1).start()

        return acc

    acc0 = ((pos, pos, pos, pos), (neg, neg, neg, neg))
    (mns, mxs) = lax.fori_loop(0, NCHUNK // 2, pair, acc0)
    mn = jnp.minimum(jnp.minimum(mns[0], mns[1]),
                     jnp.minimum(mns[2], mns[3]))
    mx = jnp.maximum(jnp.maximum(mxs[0], mxs[1]),
                     jnp.maximum(mxs[2], mxs[3]))
    res[pl.ds(0, NLANE)] = mn
    res[pl.ds(NLANE, NLANE)] = mx
    pltpu.sync_copy(res.at[pl.ds(0, NLANE)],
                    out_hbm.at[pl.ds(wid * NLANE, NLANE)])
    pltpu.sync_copy(res.at[pl.ds(NLANE, NLANE)],
                    out_hbm.at[pl.ds((NW + wid) * NLANE, NLANE)])


_sc_minmax = functools.partial(
    pl.kernel,
    out_type=jax.ShapeDtypeStruct((2 * NW * NLANE,), jnp.float32),
    mesh=plsc.VectorSubcoreMesh(core_axis_name="c", subcore_axis_name="s"),
    scratch_types=[
        pltpu.VMEM((CHUNK_R, N_COLS), jnp.float32),
        pltpu.VMEM((CHUNK_R, N_COLS), jnp.float32),
        pltpu.VMEM((2 * NLANE,), jnp.float32),
        pltpu.SemaphoreType.DMA,
        pltpu.SemaphoreType.DMA,
    ],
    compiler_params=pltpu.CompilerParams(needs_layout_passes=False),
)(_mm_body)


def _hist_body(x_hbm, mmp_hbm, out_hbm,
               buf0, buf1, mm_buf, histf, histr,
               shared, sem0, sem1):
    c = lax.axis_index("c")
    s = lax.axis_index("s")
    wid = s * NC + c
    base = wid * ROWS_W

    pltpu.sync_copy(mmp_hbm, mm_buf)
    mnv = mm_buf[pl.ds(0, NLANE)]
    mxv = mm_buf[pl.ds(NW * NLANE, NLANE)]
    for i in range(1, NW):
        mnv = jnp.minimum(mnv, mm_buf[pl.ds(i * NLANE, NLANE)])
        mxv = jnp.maximum(mxv, mm_buf[pl.ds((NW + i) * NLANE, NLANE)])
    mn_s = jnp.min(mnv)
    mx_s = jnp.max(mxv)
    mn_vec = jnp.full((NLANE,), mn_s, jnp.float32)
    w_vec = (jnp.full((NLANE,), mx_s, jnp.float32) - mn_vec) * (1.0 / NBINS)
    safe_w = jnp.where(w_vec == 0.0, jnp.float32(1.0), w_vec)
    inv_vec = jnp.float32(1.0) / safe_w

    zero16 = jnp.zeros((NLANE,), jnp.float32)
    ones16 = jnp.ones((NLANE,), jnp.float32)
    lane_off = lax.iota(jnp.int32, NLANE) * HSTRIDE
    copy_off = [lane_off + cc * (NLANE * HSTRIDE) for cc in range(NHCOPY)]

    def zbody(i, carry):
        histf[pl.ds(i * NLANE, NLANE)] = zero16
        return carry

    lax.fori_loop(0, NHCOPY * NLANE * HSTRIDE // NLANE, zbody, 0)

    def cp(ch, buf, sem):
        band = lax.rem(ch, NCHUNK)
        colh = lax.div(ch, NCHUNK)
        return pltpu.make_async_copy(
            x_hbm.at[pl.ds((base + band * CHUNK_R), CHUNK_R),
                     pl.ds(colh * HB_COLS, HB_COLS)], buf, sem)

    cp(0, buf0, sem0).start()
    cp(1, buf1, sem1).start()

    def compute(buf):
        for r in range(CHUNK_R):
            @plsc.parallel_loop(0, HB_COLS // NLANE, unroll=8)
            def _(i, _r=r):
                v = buf[_r, pl.ds(i * NLANE, NLANE)]
                t = (v - mn_vec) * inv_vec
                idx = t.astype(jnp.int32)
                plsc.addupdate_scatter(histf, [idx + lane_off], ones16)

    def pair(p, carry):
        a = 2 * p
        cp(a, buf0, sem0).wait()
        compute(buf0)

        @pl.when(a + 2 < HB_NCHUNK)
        def _():
            cp(a + 2, buf0, sem0).start()

        cp(a + 1, buf1, sem1).wait()
        compute(buf1)

        @pl.when(a + 3 < HB_NCHUNK)
        def _():
            cp(a + 3, buf1, sem1).start()

        return carry

    lax.fori_loop(0, HB_NCHUNK // 2, pair, 0)

    for cc in range(NHCOPY):
        ov = zero16
        for k in range(NBINS, HSTRIDE):
            ov = ov + plsc.load_gather(histf, [copy_off[cc] + k])
        last = plsc.load_gather(histf, [copy_off[cc] + (NBINS - 1)])
        plsc.store_scatter(histf, [copy_off[cc] + (NBINS - 1)], last + ov)

    def rbody(j, carry):
        col = j * NLANE
        acc = zero16
        for l in range(NLANE * NHCOPY):
            acc = acc + histf[pl.ds(l * HSTRIDE + col, NLANE)]
        histr[pl.ds(col, NLANE)] = acc
        return carry

    lax.fori_loop(0, NBINS // NLANE, rbody, 0)

    pltpu.sync_copy(histr, shared.at[s])
    plsc.subcore_barrier()

    STRIPE = NBINS // NS
    for l in range(NS):
        pltpu.sync_copy(shared.at[l, pl.ds(s * STRIPE, STRIPE)],
                        buf0.at[0, pl.ds(l * STRIPE, STRIPE)])

    def sbody(j, carry):
        col = j * NLANE
        acc = zero16
        for l in range(NS):
            acc = acc + buf0[0, pl.ds(l * STRIPE + col, NLANE)]
        histr[pl.ds(col, NLANE)] = acc
        return carry

    lax.fori_loop(0, STRIPE // NLANE, sbody, 0)

    pltpu.sync_copy(histr.at[pl.ds(0, STRIPE)],
                    out_hbm.at[c, pl.ds(s * STRIPE, STRIPE)])


_sc_hist = functools.partial(
    pl.kernel,
    out_type=jax.ShapeDtypeStruct((NC, NBINS), jnp.float32),
    mesh=plsc.VectorSubcoreMesh(core_axis_name="c", subcore_axis_name="s"),
    scratch_types=[
        pltpu.VMEM((CHUNK_R, HB_COLS), jnp.float32),
        pltpu.VMEM((CHUNK_R, HB_COLS), jnp.float32),
        pltpu.VMEM((2 * NW * NLANE,), jnp.float32),
        pltpu.VMEM((NHCOPY * NLANE * HSTRIDE,), jnp.float32),
        pltpu.VMEM((NBINS,), jnp.float32),
        pltpu.VMEM_SHARED((NS, NBINS), jnp.float32),
        pltpu.SemaphoreType.DMA,
        pltpu.SemaphoreType.DMA,
    ],
    compiler_params=pltpu.CompilerParams(needs_layout_passes=False),
)(_hist_body)


def _final_body(p_ref, mm_ref, h_ref, mn_ref, mx_ref):
    h_ref[...] = p_ref[0:1, :] + p_ref[1:2, :]
    mn_ref[0, 0] = jnp.min(mm_ref[0:1, :])
    mx_ref[0, 0] = jnp.max(mm_ref[1:2, :])


def _tc_finalize(partials, mmp):
    return pl.pallas_call(
        _final_body,
        out_specs=[
            pl.BlockSpec(memory_space=pltpu.VMEM),
            pl.BlockSpec(memory_space=pltpu.SMEM),
            pl.BlockSpec(memory_space=pltpu.SMEM),
        ],
        out_shape=[
            jax.ShapeDtypeStruct((1, NBINS), jnp.float32),
            jax.ShapeDtypeStruct((1, 1), jnp.float32),
            jax.ShapeDtypeStruct((1, 1), jnp.float32),
        ],
    )(partials, mmp.reshape(2, NW * NLANE))


def kernel(x):
    x2d = x.reshape(N_ROWS, N_COLS)
    mmp = _sc_minmax(x2d)
    partials = _sc_hist(x2d, mmp)
    hist2d, mn11, mx11 = _tc_finalize(partials, mmp)
    return x, hist2d.reshape(NBINS), mn11.reshape(()), mx11.reshape(())


# speedup vs baseline: 1.0799x; 1.0313x over previous
import functools

import jax
import jax.numpy as jnp
from jax import lax
from jax.experimental import pallas as pl
from jax.experimental.pallas import tpu as pltpu
from jax.experimental.pallas import tpu_sc as plsc

NBINS = 2048
HSTRIDE = NBINS + 3
NHCOPY = 1
NC = 2
NS = 16
NLANE = 16
NW = NC * NS

N_TOTAL = 2 * 8192 * 4096
N_ROWS = 16384
N_COLS = 4096
ROWS_W = N_ROWS // NW
CHUNK_R = 8
NCHUNK = ROWS_W // CHUNK_R
HB_COLS = N_COLS
HB_NCHUNK = NCHUNK


_MM_ROWS = 16384
_MM_BM = 512
_MM_GRID = _MM_ROWS // _MM_BM


def _minmax_body(x_ref, mn_ref, mx_ref):
    i = pl.program_id(0)

    @pl.when(i == 0)
    def _():
        mn_ref[0, 0] = jnp.float32(jnp.inf)
        mx_ref[0, 0] = jnp.float32(-jnp.inf)

    blk = x_ref[...]
    mn_ref[0, 0] = jnp.minimum(mn_ref[0, 0], jnp.min(blk))
    mx_ref[0, 0] = jnp.maximum(mx_ref[0, 0], jnp.max(blk))


def _tc_minmax(x2d):
    return pl.pallas_call(
        _minmax_body,
        grid=(_MM_GRID,),
        in_specs=[pl.BlockSpec((_MM_BM, 4096), lambda i: (i, 0))],
        out_specs=[
            pl.BlockSpec(memory_space=pltpu.SMEM),
            pl.BlockSpec(memory_space=pltpu.SMEM),
        ],
        out_shape=[
            jax.ShapeDtypeStruct((1, 1), jnp.float32),
            jax.ShapeDtypeStruct((1, 1), jnp.float32),
        ],
    )(x2d)


def _mm_body(x_hbm, out_hbm, buf0, buf1, res, sem0, sem1):
    c = lax.axis_index("c")
    s = lax.axis_index("s")
    wid = s * NC + c
    base = wid * ROWS_W

    def cp(ch, buf, sem):
        return pltpu.make_async_copy(
            x_hbm.at[pl.ds((base + ch * CHUNK_R), CHUNK_R), :], buf, sem)

    cp(0, buf0, sem0).start()
    cp(1, buf1, sem1).start()

    pos = jnp.full((NLANE,), jnp.inf, jnp.float32)
    neg = jnp.full((NLANE,), -jnp.inf, jnp.float32)

    def compute(buf, acc):
        for r in range(CHUNK_R):
            def body(i, a, _r=r):
                mns, mxs = a
                mns, mxs = list(mns), list(mxs)
                for k in range(4):
                    v = buf[_r, pl.ds((i * 4 + k) * NLANE, NLANE)]
                    mns[k] = jnp.minimum(mns[k], v)
                    mxs[k] = jnp.maximum(mxs[k], v)
                return tuple(mns), tuple(mxs)

            acc = lax.fori_loop(0, N_COLS // (4 * NLANE), body, acc,
                                unroll=2)
        return acc

    def pair(p, acc):
        a = 2 * p
        cp(a, buf0, sem0).wait()
        acc = compute(buf0, acc)

        @pl.when(a + 2 < NCHUNK)
        def _():
            cp(a + 2, buf0, sem0).start()

        cp(a + 1, buf1, sem1).wait()
        acc = compute(buf1, acc)

        @pl.when(a + 3 < NCHUNK)
        def _():
            cp(a + 3, buf1, sem1).start()

        return acc

    acc0 = ((pos, pos, pos, pos), (neg, neg, neg, neg))
    (mns, mxs) = lax.fori_loop(0, NCHUNK // 2, pair, acc0)
    mn = jnp.minimum(jnp.minimum(mns[0], mns[1]),
                     jnp.minimum(mns[2], mns[3]))
    mx = jnp.maximum(jnp.maximum(mxs[0], mxs[1]),
                     jnp.maximum(mxs[2], mxs[3]))
    res[pl.ds(0, NLANE)] = mn
    res[pl.ds(NLANE, NLANE)] = mx
    pltpu.sync_copy(res.at[pl.ds(0, NLANE)],
                    out_hbm.at[pl.ds(wid * NLANE, NLANE)])
    pltpu.sync_copy(res.at[pl.ds(NLANE, NLANE)],
                    out_hbm.at[pl.ds((NW + wid) * NLANE, NLANE)])


_sc_minmax = functools.partial(
    pl.kernel,
    out_type=jax.ShapeDtypeStruct((2 * NW * NLANE,), jnp.float32),
    mesh=plsc.VectorSubcoreMesh(core_axis_name="c", subcore_axis_name="s"),
    scratch_types=[
        pltpu.VMEM((CHUNK_R, N_COLS), jnp.float32),
        pltpu.VMEM((CHUNK_R, N_COLS), jnp.float32),
        pltpu.VMEM((2 * NLANE,), jnp.float32),
        pltpu.SemaphoreType.DMA,
        pltpu.SemaphoreType.DMA,
    ],
    compiler_params=pltpu.CompilerParams(needs_layout_passes=False),
)(_mm_body)


def _hist_body(x_hbm, mmp_hbm, out_hbm,
               buf0, buf1, mm_buf, histf, histr,
               shared, sem0, sem1):
    c = lax.axis_index("c")
    s = lax.axis_index("s")
    wid = s * NC + c
    base = wid * ROWS_W

    pltpu.sync_copy(mmp_hbm, mm_buf)
    mn_vec = mm_buf[pl.ds(0, NLANE)]
    mx_vec = mm_buf[pl.ds(NLANE, NLANE)]
    w_vec = (mx_vec - mn_vec) * (1.0 / NBINS)
    safe_w = jnp.where(w_vec == 0.0, jnp.float32(1.0), w_vec)
    inv_vec = jnp.float32(1.0) / safe_w

    zero16 = jnp.zeros((NLANE,), jnp.float32)
    ones16 = jnp.ones((NLANE,), jnp.float32)
    lane_off = lax.iota(jnp.int32, NLANE) * HSTRIDE
    copy_off = [lane_off + cc * (NLANE * HSTRIDE) for cc in range(NHCOPY)]

    def zbody(i, carry):
        histf[pl.ds(i * NLANE, NLANE)] = zero16
        return carry

    lax.fori_loop(0, NHCOPY * NLANE * HSTRIDE // NLANE, zbody, 0)

    def cp(ch, buf, sem):
        band = lax.rem(ch, NCHUNK)
        colh = lax.div(ch, NCHUNK)
        return pltpu.make_async_copy(
            x_hbm.at[pl.ds((base + band * CHUNK_R), CHUNK_R),
                     pl.ds(colh * HB_COLS, HB_COLS)], buf, sem)

    cp(0, buf0, sem0).start()
    cp(1, buf1, sem1).start()

    def compute(buf):
        for r in range(CHUNK_R):
            @plsc.parallel_loop(0, HB_COLS // NLANE, unroll=8)
            def _(i, _r=r):
                v = buf[_r, pl.ds(i * NLANE, NLANE)]
                t = (v - mn_vec) * inv_vec
                idx = t.astype(jnp.int32)
                plsc.addupdate_scatter(histf, [idx + lane_off], ones16)

    def pair(p, carry):
        a = 2 * p
        cp(a, buf0, sem0).wait()
        compute(buf0)

        @pl.when(a + 2 < HB_NCHUNK)
        def _():
            cp(a + 2, buf0, sem0).start()

        cp(a + 1, buf1, sem1).wait()
        compute(buf1)

        @pl.when(a + 3 < HB_NCHUNK)
        def _():
            cp(a + 3, buf1, sem1).start()

        return carry

    lax.fori_loop(0, HB_NCHUNK // 2, pair, 0)

    for cc in range(NHCOPY):
        ov = zero16
        for k in range(NBINS, HSTRIDE):
            ov = ov + plsc.load_gather(histf, [copy_off[cc] + k])
        last = plsc.load_gather(histf, [copy_off[cc] + (NBINS - 1)])
        plsc.store_scatter(histf, [copy_off[cc] + (NBINS - 1)], last + ov)

    def rbody(j, carry):
        col = j * NLANE
        acc = zero16
        for l in range(NLANE * NHCOPY):
            acc = acc + histf[pl.ds(l * HSTRIDE + col, NLANE)]
        histr[pl.ds(col, NLANE)] = acc
        return carry

    lax.fori_loop(0, NBINS // NLANE, rbody, 0)

    pltpu.sync_copy(histr, shared.at[s])
    plsc.subcore_barrier()

    STRIPE = NBINS // NS
    for l in range(NS):
        pltpu.sync_copy(shared.at[l, pl.ds(s * STRIPE, STRIPE)],
                        buf0.at[0, pl.ds(l * STRIPE, STRIPE)])

    def sbody(j, carry):
        col = j * NLANE
        acc = zero16
        for l in range(NS):
            acc = acc + buf0[0, pl.ds(l * STRIPE + col, NLANE)]
        histr[pl.ds(col, NLANE)] = acc
        return carry

    lax.fori_loop(0, STRIPE // NLANE, sbody, 0)

    pltpu.sync_copy(histr.at[pl.ds(0, STRIPE)],
                    out_hbm.at[c, pl.ds(s * STRIPE, STRIPE)])


_sc_hist = functools.partial(
    pl.kernel,
    out_type=jax.ShapeDtypeStruct((NC, NBINS), jnp.float32),
    mesh=plsc.VectorSubcoreMesh(core_axis_name="c", subcore_axis_name="s"),
    scratch_types=[
        pltpu.VMEM((CHUNK_R, HB_COLS), jnp.float32),
        pltpu.VMEM((CHUNK_R, HB_COLS), jnp.float32),
        pltpu.VMEM((2 * NLANE,), jnp.float32),
        pltpu.VMEM((NHCOPY * NLANE * HSTRIDE,), jnp.float32),
        pltpu.VMEM((NBINS,), jnp.float32),
        pltpu.VMEM_SHARED((NS, NBINS), jnp.float32),
        pltpu.SemaphoreType.DMA,
        pltpu.SemaphoreType.DMA,
    ],
    compiler_params=pltpu.CompilerParams(needs_layout_passes=False),
)(_hist_body)


def _final_body(p_ref, mm_ref, h_ref, mn_ref, mx_ref):
    h_ref[...] = p_ref[0:1, :] + p_ref[1:2, :]
    mn_ref[0, 0] = mm_ref[0, 0]
    mx_ref[0, 0] = mm_ref[0, 1]


def _tc_finalize(partials, mmp):
    return pl.pallas_call(
        _final_body,
        out_specs=[
            pl.BlockSpec(memory_space=pltpu.VMEM),
            pl.BlockSpec(memory_space=pltpu.SMEM),
            pl.BlockSpec(memory_space=pltpu.SMEM),
        ],
        out_shape=[
            jax.ShapeDtypeStruct((1, NBINS), jnp.float32),
            jax.ShapeDtypeStruct((1, 1), jnp.float32),
            jax.ShapeDtypeStruct((1, 1), jnp.float32),
        ],
        in_specs=[
            pl.BlockSpec(memory_space=pltpu.VMEM),
            pl.BlockSpec(memory_space=pltpu.SMEM),
        ],
    )(partials, mmp)


def kernel(x):
    x2d = x.reshape(N_ROWS, N_COLS)
    mn11, mx11 = _tc_minmax(x2d)
    mm2 = jnp.concatenate([mn11, mx11], axis=1)
    mm32 = jnp.broadcast_to(mm2.reshape(2, 1), (2, NLANE)).reshape(2 * NLANE)
    partials = _sc_hist(x2d, mm32)
    hist2d, mn, mx = _tc_finalize(partials, mm2)
    return x, hist2d.reshape(NBINS), mn.reshape(()), mx.reshape(())


# speedup vs baseline: 1.0825x; 1.0024x over previous
import functools

import jax
import jax.numpy as jnp
from jax import lax
from jax.experimental import pallas as pl
from jax.experimental.pallas import tpu as pltpu
from jax.experimental.pallas import tpu_sc as plsc

NBINS = 2048
HSTRIDE = NBINS + 3
NHCOPY = 1
NC = 2
NS = 16
NLANE = 16
NW = NC * NS

N_TOTAL = 2 * 8192 * 4096
N_ROWS = 16384
N_COLS = 4096
ROWS_W = N_ROWS // NW
CHUNK_R = 8
NCHUNK = ROWS_W // CHUNK_R
HB_COLS = N_COLS
HB_NCHUNK = NCHUNK


_MM_ROWS = 16384
_MM_BM = 512
_MM_GRID = _MM_ROWS // _MM_BM


def _minmax_body(x_ref, mn_ref, mx_ref):
    i = pl.program_id(0)

    @pl.when(i == 0)
    def _():
        mn_ref[0, 0] = jnp.float32(jnp.inf)
        mx_ref[0, 0] = jnp.float32(-jnp.inf)

    blk = x_ref[...]
    mn_ref[0, 0] = jnp.minimum(mn_ref[0, 0], jnp.min(blk))
    mx_ref[0, 0] = jnp.maximum(mx_ref[0, 0], jnp.max(blk))


def _tc_minmax(x2d):
    return pl.pallas_call(
        _minmax_body,
        grid=(_MM_GRID,),
        in_specs=[pl.BlockSpec((_MM_BM, 4096), lambda i: (i, 0))],
        out_specs=[
            pl.BlockSpec(memory_space=pltpu.SMEM),
            pl.BlockSpec(memory_space=pltpu.SMEM),
        ],
        out_shape=[
            jax.ShapeDtypeStruct((1, 1), jnp.float32),
            jax.ShapeDtypeStruct((1, 1), jnp.float32),
        ],
    )(x2d)


def _hist_body(x_hbm, mmp_hbm, out_hbm,
               buf0, buf1, mm_buf, histf, histr,
               shared, sem0, sem1):
    c = lax.axis_index("c")
    s = lax.axis_index("s")
    wid = s * NC + c
    base = wid * ROWS_W

    pltpu.sync_copy(mmp_hbm, mm_buf)
    mn_vec = mm_buf[pl.ds(0, NLANE)]
    mx_vec = mm_buf[pl.ds(NLANE, NLANE)]
    w_vec = (mx_vec - mn_vec) * (1.0 / NBINS)
    safe_w = jnp.where(w_vec == 0.0, jnp.float32(1.0), w_vec)
    inv_vec = jnp.float32(1.0) / safe_w

    zero16 = jnp.zeros((NLANE,), jnp.float32)
    ones16 = jnp.ones((NLANE,), jnp.float32)
    lane_off = lax.iota(jnp.int32, NLANE) * HSTRIDE
    copy_off = [lane_off + cc * (NLANE * HSTRIDE) for cc in range(NHCOPY)]

    def zbody(i, carry):
        histf[pl.ds(i * NLANE, NLANE)] = zero16
        return carry

    lax.fori_loop(0, NHCOPY * NLANE * HSTRIDE // NLANE, zbody, 0)

    def cp(ch, buf, sem):
        band = lax.rem(ch, NCHUNK)
        colh = lax.div(ch, NCHUNK)
        return pltpu.make_async_copy(
            x_hbm.at[pl.ds((base + band * CHUNK_R), CHUNK_R),
                     pl.ds(colh * HB_COLS, HB_COLS)], buf, sem)

    cp(0, buf0, sem0).start()
    cp(1, buf1, sem1).start()

    def compute(buf):
        for r in range(CHUNK_R):
            @plsc.parallel_loop(0, HB_COLS // NLANE, unroll=8)
            def _(i, _r=r):
                v = buf[_r, pl.ds(i * NLANE, NLANE)]
                t = (v - mn_vec) * inv_vec
                idx = t.astype(jnp.int32)
                plsc.addupdate_scatter(histf, [idx + lane_off], ones16)

    def pair(p, carry):
        a = 2 * p
        cp(a, buf0, sem0).wait()
        compute(buf0)

        @pl.when(a + 2 < HB_NCHUNK)
        def _():
            cp(a + 2, buf0, sem0).start()

        cp(a + 1, buf1, sem1).wait()
        compute(buf1)

        @pl.when(a + 3 < HB_NCHUNK)
        def _():
            cp(a + 3, buf1, sem1).start()

        return carry

    lax.fori_loop(0, HB_NCHUNK // 2, pair, 0)

    for cc in range(NHCOPY):
        ov = zero16
        for k in range(NBINS, HSTRIDE):
            ov = ov + plsc.load_gather(histf, [copy_off[cc] + k])
        last = plsc.load_gather(histf, [copy_off[cc] + (NBINS - 1)])
        plsc.store_scatter(histf, [copy_off[cc] + (NBINS - 1)], last + ov)

    def rbody(j, carry):
        col = j * NLANE
        acc = zero16
        for l in range(NLANE * NHCOPY):
            acc = acc + histf[pl.ds(l * HSTRIDE + col, NLANE)]
        histr[pl.ds(col, NLANE)] = acc
        return carry

    lax.fori_loop(0, NBINS // NLANE, rbody, 0)

    pltpu.sync_copy(histr, shared.at[s])
    plsc.subcore_barrier()

    STRIPE = NBINS // NS
    for l in range(NS):
        pltpu.sync_copy(shared.at[l, pl.ds(s * STRIPE, STRIPE)],
                        buf0.at[0, pl.ds(l * STRIPE, STRIPE)])

    def sbody(j, carry):
        col = j * NLANE
        acc = zero16
        for l in range(NS):
            acc = acc + buf0[0, pl.ds(l * STRIPE + col, NLANE)]
        histr[pl.ds(col, NLANE)] = acc
        return carry

    lax.fori_loop(0, STRIPE // NLANE, sbody, 0)

    pltpu.sync_copy(histr.at[pl.ds(0, STRIPE)],
                    out_hbm.at[c, pl.ds(s * STRIPE, STRIPE)])


_sc_hist = functools.partial(
    pl.kernel,
    out_type=jax.ShapeDtypeStruct((NC, NBINS), jnp.float32),
    mesh=plsc.VectorSubcoreMesh(core_axis_name="c", subcore_axis_name="s"),
    scratch_types=[
        pltpu.VMEM((CHUNK_R, HB_COLS), jnp.float32),
        pltpu.VMEM((CHUNK_R, HB_COLS), jnp.float32),
        pltpu.VMEM((2 * NLANE,), jnp.float32),
        pltpu.VMEM((NHCOPY * NLANE * HSTRIDE,), jnp.float32),
        pltpu.VMEM((NBINS,), jnp.float32),
        pltpu.VMEM_SHARED((NS, NBINS), jnp.float32),
        pltpu.SemaphoreType.DMA,
        pltpu.SemaphoreType.DMA,
    ],
    compiler_params=pltpu.CompilerParams(needs_layout_passes=False),
)(_hist_body)


def _final_body(p_ref, mm_ref, h_ref, mn_ref, mx_ref):
    h_ref[...] = p_ref[0:1, :] + p_ref[1:2, :]
    mn_ref[0, 0] = mm_ref[0, 0]
    mx_ref[0, 0] = mm_ref[0, 1]


def _tc_finalize(partials, mmp):
    return pl.pallas_call(
        _final_body,
        out_specs=[
            pl.BlockSpec(memory_space=pltpu.VMEM),
            pl.BlockSpec(memory_space=pltpu.SMEM),
            pl.BlockSpec(memory_space=pltpu.SMEM),
        ],
        out_shape=[
            jax.ShapeDtypeStruct((1, NBINS), jnp.float32),
            jax.ShapeDtypeStruct((1, 1), jnp.float32),
            jax.ShapeDtypeStruct((1, 1), jnp.float32),
        ],
        in_specs=[
            pl.BlockSpec(memory_space=pltpu.VMEM),
            pl.BlockSpec(memory_space=pltpu.SMEM),
        ],
    )(partials, mmp)


def kernel(x):
    x2d = x.reshape(N_ROWS, N_COLS)
    mn11, mx11 = _tc_minmax(x2d)
    mm2 = jnp.concatenate([mn11, mx11], axis=1)
    mm32 = jnp.broadcast_to(mm2.reshape(2, 1), (2, NLANE)).reshape(2 * NLANE)
    partials = _sc_hist(x2d, mm32)
    hist2d, mn, mx = _tc_finalize(partials, mm2)
    return x, hist2d.reshape(NBINS), mn.reshape(()), mx.reshape(())
